# baseline (speedup 1.0000x reference)
import functools

import jax
import jax.numpy as jnp
from jax import lax
from jax.experimental import pallas as pl
from jax.experimental.pallas import tpu as pltpu
from jax.experimental.pallas import tpu_sc as plsc

_VOCAB = 1000000
_D = 64
_P = 128
_B = 1024
_DOC_LEN = 200
_QUERY_LEN = 50

_NC = 2
_NS = 16
_NW = _NC * _NS
_BPW = _B // _NW
_DCH = 100
_DCHUNKS = _DOC_LEN // _DCH


def _pool_rows(rows_ref, n_rows, acc):
    def body(j, a):
        a0, a1, a2, a3 = a
        r = 4 * j
        for _ in range(4):
            a0 = a0 + rows_ref[r, pl.ds(0, 16)]
            a1 = a1 + rows_ref[r, pl.ds(16, 16)]
            a2 = a2 + rows_ref[r, pl.ds(32, 16)]
            a3 = a3 + rows_ref[r, pl.ds(48, 16)]
            r = r + 1
        return (a0, a1, a2, a3)

    return lax.fori_loop(0, n_rows // 4, body, acc, unroll=False)


def _sc_pool_kernel(doc_idx_hbm, q_idx_hbm, table_hbm, d_out_hbm, q_out_hbm,
                    didx_v, qidx_v, rows_v, dacc_v, qacc_v, sem):
    wid = lax.axis_index("s") * _NC + lax.axis_index("c")

    pltpu.sync_copy(doc_idx_hbm.at[pl.ds(wid * _BPW * _DCHUNKS, _BPW * _DCHUNKS)],
                    didx_v)
    pltpu.sync_copy(q_idx_hbm.at[pl.ds(wid * _BPW, _BPW)], qidx_v)

    zero = jnp.zeros((16,), jnp.float32)

    def doc_item(i, carry):
        acc = (zero, zero, zero, zero)
        for c in range(_DCHUNKS):
            pltpu.async_copy(table_hbm.at[didx_v.at[i * _DCHUNKS + c]],
                             rows_v.at[pl.ds(0, _DCH)], sem).wait()
            acc = _pool_rows(rows_v, _DCH, acc)
        a0, a1, a2, a3 = acc
        dacc_v[i, pl.ds(0, 16)] = a0
        dacc_v[i, pl.ds(16, 16)] = a1
        dacc_v[i, pl.ds(32, 16)] = a2
        dacc_v[i, pl.ds(48, 16)] = a3
        return carry

    lax.fori_loop(0, _BPW, doc_item, 0, unroll=False)

    def q_item(i, carry):
        pltpu.async_copy(table_hbm.at[qidx_v.at[i]],
                         rows_v.at[pl.ds(0, _QUERY_LEN)], sem).wait()
        acc = _pool_rows(rows_v, 48, (zero, zero, zero, zero))
        a0, a1, a2, a3 = acc
        for r in (48, 49):
            a0 = a0 + rows_v[r, pl.ds(0, 16)]
            a1 = a1 + rows_v[r, pl.ds(16, 16)]
            a2 = a2 + rows_v[r, pl.ds(32, 16)]
            a3 = a3 + rows_v[r, pl.ds(48, 16)]
        qacc_v[i, pl.ds(0, 16)] = a0
        qacc_v[i, pl.ds(16, 16)] = a1
        qacc_v[i, pl.ds(32, 16)] = a2
        qacc_v[i, pl.ds(48, 16)] = a3
        return carry

    lax.fori_loop(0, _BPW, q_item, 0, unroll=False)

    pltpu.sync_copy(dacc_v, d_out_hbm.at[pl.ds(wid * _BPW, _BPW)])
    pltpu.sync_copy(qacc_v, q_out_hbm.at[pl.ds(wid * _BPW, _BPW)])


def _sc_pool(doc_idx, q_idx, table):
    mesh = plsc.VectorSubcoreMesh(core_axis_name="c", subcore_axis_name="s")
    fn = functools.partial(
        pl.kernel,
        mesh=mesh,
        compiler_params=pltpu.CompilerParams(use_tc_tiling_on_sc=False),
        out_type=[
            jax.ShapeDtypeStruct((_B, _D), jnp.float32),
            jax.ShapeDtypeStruct((_B, _D), jnp.float32),
        ],
        scratch_types=[
            pltpu.VMEM((_BPW * _DCHUNKS, _DCH), jnp.int32),
            pltpu.VMEM((_BPW, _QUERY_LEN), jnp.int32),
            pltpu.VMEM((_DCH, _D), jnp.float32),
            pltpu.VMEM((_BPW, _D), jnp.float32),
            pltpu.VMEM((_BPW, _D), jnp.float32),
            pltpu.SemaphoreType.DMA,
        ],
    )(_sc_pool_kernel)
    return fn(doc_idx, q_idx, table)


def _tc_head_kernel(d_ref, q_ref, dw1_ref, db1_ref, dw2_ref, db2_ref,
                    qw1_ref, qb1_ref, qw2_ref, qb2_ref, out_ref):
    def dot_t(a, w):
        return lax.dot_general(a, w, (((1,), (1,)), ((), ())),
                               preferred_element_type=jnp.float32)

    d = d_ref[...] * (1.0 / _DOC_LEN)
    q = q_ref[...] * (1.0 / _QUERY_LEN)
    dh = jnp.maximum(dot_t(d, dw1_ref[...]) + db1_ref[...], 0.0)
    dp = dot_t(dh, dw2_ref[...]) + db2_ref[...]
    qh = jnp.maximum(dot_t(q, qw1_ref[...]) + qb1_ref[...], 0.0)
    qp = dot_t(qh, qw2_ref[...]) + qb2_ref[...]
    dn = jnp.maximum(jnp.sqrt(jnp.sum(dp * dp, axis=1, keepdims=True)), 1e-8)
    qn = jnp.maximum(jnp.sqrt(jnp.sum(qp * qp, axis=1, keepdims=True)), 1e-8)
    out_ref[...] = jnp.sum(dp * qp, axis=1, keepdims=True) / (dn * qn)


def _tc_head(d_sum, q_sum, d_w1, d_b1, d_w2, d_b2, q_w1, q_b1, q_w2, q_b2):
    return pl.pallas_call(
        _tc_head_kernel,
        out_shape=jax.ShapeDtypeStruct((_B, 1), jnp.float32),
    )(d_sum, q_sum, d_w1, d_b1.reshape(1, _P), d_w2, d_b2.reshape(1, _P),
      q_w1, q_b1.reshape(1, _D), q_w2, q_b2.reshape(1, _P))


def kernel(doc_ids, query_ids, table, d_w1, d_b1, d_w2, d_b2,
           q_w1, q_b1, q_w2, q_b2):
    doc_idx = doc_ids.astype(jnp.int32).reshape(_B * _DCHUNKS, _DCH)
    q_idx = query_ids.astype(jnp.int32)
    d_sum, q_sum = _sc_pool(doc_idx, q_idx, table)
    sim = _tc_head(d_sum, q_sum, d_w1, d_b1, d_w2, d_b2,
                   q_w1, q_b1, q_w2, q_b2)
    return sim.reshape(_B)


# speedup vs baseline: 1.0948x; 1.0948x over previous
import functools

import jax
import jax.numpy as jnp
from jax import lax
from jax.experimental import pallas as pl
from jax.experimental.pallas import tpu as pltpu
from jax.experimental.pallas import tpu_sc as plsc

_VOCAB = 1000000
_D = 64
_P = 128
_B = 1024
_DOC_LEN = 200
_QUERY_LEN = 50

_NC = 2
_NS = 16
_NW = _NC * _NS
_BPW = _B // _NW
_DCH = 100
_DCHUNKS = _DOC_LEN // _DCH


def _pool_rows(rows_ref, n_rows, acc):
    def body(j, a):
        a0, a1, a2, a3 = a
        r = 4 * j
        for _ in range(4):
            a0 = a0 + rows_ref[r, pl.ds(0, 16)]
            a1 = a1 + rows_ref[r, pl.ds(16, 16)]
            a2 = a2 + rows_ref[r, pl.ds(32, 16)]
            a3 = a3 + rows_ref[r, pl.ds(48, 16)]
            r = r + 1
        return (a0, a1, a2, a3)

    return lax.fori_loop(0, n_rows // 4, body, acc, unroll=False)


_NBUF = 3


def _store_acc(acc_ref, i, acc):
    a0, a1, a2, a3 = acc
    acc_ref[i, pl.ds(0, 16)] = a0
    acc_ref[i, pl.ds(16, 16)] = a1
    acc_ref[i, pl.ds(32, 16)] = a2
    acc_ref[i, pl.ds(48, 16)] = a3


def _sc_pool_kernel(doc_idx_hbm, q_idx_hbm, table_hbm, d_out_hbm, q_out_hbm,
                    didx_v, qidx_v, rows0, rows1, rows2, dacc_v, qacc_v,
                    sem0, sem1, sem2):
    wid = lax.axis_index("s") * _NC + lax.axis_index("c")
    rows = (rows0, rows1, rows2)
    sems = (sem0, sem1, sem2)

    pltpu.sync_copy(doc_idx_hbm.at[pl.ds(wid * _BPW * _DCHUNKS, _BPW * _DCHUNKS)],
                    didx_v)
    pltpu.sync_copy(q_idx_hbm.at[pl.ds(wid * _BPW, _BPW)], qidx_v)

    zero = jnp.zeros((16,), jnp.float32)
    z4 = (zero, zero, zero, zero)

    def d_start(i, b):
        return [
            pltpu.async_copy(table_hbm.at[didx_v.at[_DCHUNKS * i + c]],
                             rows[b].at[pl.ds(c * _DCH, _DCH)], sems[b])
            for c in range(_DCHUNKS)
        ]

    descs = {}
    for i in range(_NBUF):
        descs[i] = d_start(i, i % _NBUF)
    for i in range(_BPW):
        b = i % _NBUF
        for dsc in descs.pop(i):
            dsc.wait()
        acc = _pool_rows(rows[b], _DOC_LEN, z4)
        _store_acc(dacc_v, i, acc)
        if i + _NBUF < _BPW:
            descs[i + _NBUF] = d_start(i + _NBUF, b)

    pltpu.sync_copy(dacc_v, d_out_hbm.at[pl.ds(wid * _BPW, _BPW)])

    def q_start(i, b):
        return pltpu.async_copy(table_hbm.at[qidx_v.at[i]],
                                rows[b].at[pl.ds(0, _QUERY_LEN)], sems[b])

    descs = {}
    for i in range(_NBUF):
        descs[i] = q_start(i, i % _NBUF)
    for i in range(_BPW):
        b = i % _NBUF
        descs.pop(i).wait()
        a0, a1, a2, a3 = _pool_rows(rows[b], 48, z4)
        for r in (48, 49):
            a0 = a0 + rows[b][r, pl.ds(0, 16)]
            a1 = a1 + rows[b][r, pl.ds(16, 16)]
            a2 = a2 + rows[b][r, pl.ds(32, 16)]
            a3 = a3 + rows[b][r, pl.ds(48, 16)]
        _store_acc(qacc_v, i, (a0, a1, a2, a3))
        if i + _NBUF < _BPW:
            descs[i + _NBUF] = q_start(i + _NBUF, b)

    pltpu.sync_copy(qacc_v, q_out_hbm.at[pl.ds(wid * _BPW, _BPW)])


def _sc_pool(doc_idx, q_idx, table):
    mesh = plsc.VectorSubcoreMesh(core_axis_name="c", subcore_axis_name="s")
    fn = functools.partial(
        pl.kernel,
        mesh=mesh,
        compiler_params=pltpu.CompilerParams(use_tc_tiling_on_sc=False),
        out_type=[
            jax.ShapeDtypeStruct((_B, _D), jnp.float32),
            jax.ShapeDtypeStruct((_B, _D), jnp.float32),
        ],
        scratch_types=[
            pltpu.VMEM((_BPW * _DCHUNKS, _DCH), jnp.int32),
            pltpu.VMEM((_BPW, _QUERY_LEN), jnp.int32),
            pltpu.VMEM((_DOC_LEN, _D), jnp.float32),
            pltpu.VMEM((_DOC_LEN, _D), jnp.float32),
            pltpu.VMEM((_DOC_LEN, _D), jnp.float32),
            pltpu.VMEM((_BPW, _D), jnp.float32),
            pltpu.VMEM((_BPW, _D), jnp.float32),
            pltpu.SemaphoreType.DMA,
            pltpu.SemaphoreType.DMA,
            pltpu.SemaphoreType.DMA,
        ],
    )(_sc_pool_kernel)
    return fn(doc_idx, q_idx, table)


def _tc_head_kernel(d_ref, q_ref, dw1_ref, db1_ref, dw2_ref, db2_ref,
                    qw1_ref, qb1_ref, qw2_ref, qb2_ref, out_ref):
    def dot_t(a, w):
        return lax.dot_general(a, w, (((1,), (1,)), ((), ())),
                               preferred_element_type=jnp.float32)

    d = d_ref[...] * (1.0 / _DOC_LEN)
    q = q_ref[...] * (1.0 / _QUERY_LEN)
    dh = jnp.maximum(dot_t(d, dw1_ref[...]) + db1_ref[...], 0.0)
    dp = dot_t(dh, dw2_ref[...]) + db2_ref[...]
    qh = jnp.maximum(dot_t(q, qw1_ref[...]) + qb1_ref[...], 0.0)
    qp = dot_t(qh, qw2_ref[...]) + qb2_ref[...]
    dn = jnp.maximum(jnp.sqrt(jnp.sum(dp * dp, axis=1, keepdims=True)), 1e-8)
    qn = jnp.maximum(jnp.sqrt(jnp.sum(qp * qp, axis=1, keepdims=True)), 1e-8)
    out_ref[...] = jnp.sum(dp * qp, axis=1, keepdims=True) / (dn * qn)


def _tc_head(d_sum, q_sum, d_w1, d_b1, d_w2, d_b2, q_w1, q_b1, q_w2, q_b2):
    return pl.pallas_call(
        _tc_head_kernel,
        out_shape=jax.ShapeDtypeStruct((_B, 1), jnp.float32),
    )(d_sum, q_sum, d_w1, d_b1.reshape(1, _P), d_w2, d_b2.reshape(1, _P),
      q_w1, q_b1.reshape(1, _D), q_w2, q_b2.reshape(1, _P))


def kernel(doc_ids, query_ids, table, d_w1, d_b1, d_w2, d_b2,
           q_w1, q_b1, q_w2, q_b2):
    doc_idx = doc_ids.astype(jnp.int32).reshape(_B * _DCHUNKS, _DCH)
    q_idx = query_ids.astype(jnp.int32)
    d_sum, q_sum = _sc_pool(doc_idx, q_idx, table)
    sim = _tc_head(d_sum, q_sum, d_w1, d_b1, d_w2, d_b2,
                   q_w1, q_b1, q_w2, q_b2)
    return sim.reshape(_B)
